# baseline (speedup 1.0000x reference)
import functools

import jax
import jax.numpy as jnp
from jax.experimental import pallas as pl
from jax.experimental.pallas import tpu as pltpu

_N = 50
_F = 128
_D = 32


def _prep_body(users_ref, gamma_ref, beta_ref, Wue_ref, bue_ref, Wpe_ref,
               bpe_ref, Wg0_ref, Wu0_ref, bu0_ref, Wu1_ref, bu1_ref,
               Wp0_ref, bp0_ref, Wp1_ref, bp1_ref,
               as0_ref, ad0_ref, Wg1_ref, as1_ref, ad1_ref,
               mean_ref, ginv_ref, t_ref, vpost_ref, cu_ref, wp_ref, cp_ref,
               Wx0_ref, bg0_ref, ce0_ref, Wx1_ref):
    u = users_ref[...]
    mean = jnp.mean(u, axis=0, keepdims=True)
    var = jnp.mean((u - mean) * (u - mean), axis=0, keepdims=True)
    ginv = gamma_ref[...] * jax.lax.rsqrt(var + 1e-5)
    mean_ref[...] = mean
    ginv_ref[...] = ginv

    dot = functools.partial(jnp.dot, preferred_element_type=jnp.float32)
    wu = dot(Wu0_ref[...], Wu1_ref[...])
    wu_top = wu[0:_D, :]
    t = dot(Wue_ref[...], wu_top)
    t_ref[...] = t
    vpost_ref[...] = wu[_D:2 * _D, :]
    cu_ref[...] = (dot(bu0_ref[...], Wu1_ref[...]) + bu1_ref[...]
                   + dot(beta_ref[...], t) + dot(bue_ref[...], wu_top))
    wp = dot(Wp0_ref[...], Wp1_ref[...])
    wp_ref[...] = wp
    cp_ref[...] = dot(bp0_ref[...], Wp1_ref[...]) + bp1_ref[...]
    Wg0e = dot(Wpe_ref[...], Wg0_ref[...])
    bg0 = dot(bpe_ref[...], Wg0_ref[...])
    Wx0_ref[...] = jnp.concatenate(
        [Wg0e, dot(Wg0e, as0_ref[...]), dot(Wg0e, ad0_ref[...])],
        axis=1)
    bg0_ref[...] = bg0
    ce0_ref[...] = dot(bg0, as0_ref[...] + ad0_ref[...])
    Wg1 = Wg1_ref[...]
    Wx1_ref[...] = jnp.concatenate(
        [Wg1, dot(Wg1, as1_ref[...]), dot(Wg1, ad1_ref[...])],
        axis=1)


def _bdot(a, b):
    return jax.lax.dot_general(
        a, b, (((2,), (1,)), ((0,), (0,))),
        preferred_element_type=jnp.float32)


def _attend(hx, adj, ce, bias):
    hw = hx[:, :, 0:_D] + bias
    es = hx[:, :, _D:_D + 1]
    ed = jnp.swapaxes(hx[:, :, _D + 1:_D + 2], 1, 2)
    e = es + ed + ce
    e = jnp.where(e >= 0, e, 0.2 * e)
    e = jnp.where(adj > 0, e, -1e9)
    m = jnp.max(e, axis=2, keepdims=True)
    p = jnp.exp(e - m)
    s = jnp.sum(p, axis=2, keepdims=True)
    out = _bdot(p, hw) / s
    return jnp.where(out > 0, out, jnp.exp(out) - 1.0)


def _main_body(posts_ref, adj_ref, users_ref, mean_ref, ginv_ref, t_ref,
               vpost_ref, cu_ref, wp_ref, cp_ref, Wx0_ref, bg0_ref, ce0_ref,
               Wx1_ref, ul_ref, plab_ref):
    cb = posts_ref.shape[0]
    posts = posts_ref[...]
    adj = adj_ref[...]

    Wx0_b = jnp.broadcast_to(Wx0_ref[...][None], (cb, _F, _D + 2))
    hx0 = _bdot(posts, Wx0_b)
    h1 = _attend(hx0, adj, ce0_ref[...][None], bg0_ref[...][None])

    Wx1_b = jnp.broadcast_to(Wx1_ref[...][None], (cb, _D, _D + 2))
    hx1 = _bdot(h1, Wx1_b)
    zero = jnp.zeros((1, 1, 1), jnp.float32)
    pe = _attend(hx1, adj, zero, zero)

    wp_b = jnp.broadcast_to(wp_ref[...][None], (cb, _D, 1))
    pco = _bdot(pe, wp_b) + cp_ref[...][None]
    plab_ref[...] = jax.nn.sigmoid(pco)

    mp = jnp.max(pe, axis=1)
    un = (users_ref[...] - mean_ref[...]) * ginv_ref[...]
    uco = (jnp.dot(un, t_ref[...], preferred_element_type=jnp.float32)
           + jnp.dot(mp, vpost_ref[...], preferred_element_type=jnp.float32)
           + cu_ref[...])
    ul_ref[...] = jax.nn.sigmoid(uco)


def kernel(users, posts, post_adjs, up_masking, bn_gamma, bn_beta,
           W_user_enc, b_user_enc, W_post_enc, b_post_enc,
           W_gat0, a_src0, a_dst0, W_gat1, a_src1, a_dst1,
           W_pcls0, b_pcls0, W_pcls1, b_pcls1,
           W_ucls0, b_ucls0, W_ucls1, b_ucls1):
    B, F = users.shape
    N = posts.shape[1]
    D = W_gat0.shape[0]

    row = lambda v: v.reshape(1, -1)
    col = lambda v: v.reshape(-1, 1)
    f32 = jnp.float32

    prep_outs = (
        jax.ShapeDtypeStruct((1, F), f32),
        jax.ShapeDtypeStruct((1, F), f32),
        jax.ShapeDtypeStruct((F, 1), f32),
        jax.ShapeDtypeStruct((D, 1), f32),
        jax.ShapeDtypeStruct((1, 1), f32),
        jax.ShapeDtypeStruct((D, 1), f32),
        jax.ShapeDtypeStruct((1, 1), f32),
        jax.ShapeDtypeStruct((F, D + 2), f32),
        jax.ShapeDtypeStruct((1, D), f32),
        jax.ShapeDtypeStruct((1, 1), f32),
        jax.ShapeDtypeStruct((D, D + 2), f32),
    )
    (mean, ginv, t, vpost, cu, wp, cp, Wx0, bg0, ce0, Wx1) = pl.pallas_call(
        _prep_body, out_shape=prep_outs)(
            users, row(bn_gamma), row(bn_beta), W_user_enc, row(b_user_enc),
            W_post_enc, row(b_post_enc), W_gat0, W_ucls0, row(b_ucls0),
            W_ucls1, row(b_ucls1), W_pcls0, row(b_pcls0), W_pcls1,
            row(b_pcls1), col(a_src0), col(a_dst0), W_gat1,
            col(a_src1), col(a_dst1))

    CB = 64
    grid = (B // CB,)
    full = lambda shape: pl.BlockSpec(shape, lambda i: (0,) * len(shape))
    in_specs = [
        pl.BlockSpec((CB, N, F), lambda i: (i, 0, 0)),
        pl.BlockSpec((CB, N, N), lambda i: (i, 0, 0)),
        pl.BlockSpec((CB, F), lambda i: (i, 0)),
        full((1, F)), full((1, F)), full((F, 1)), full((D, 1)),
        full((1, 1)), full((D, 1)), full((1, 1)), full((F, D + 2)),
        full((1, D)), full((1, 1)), full((D, D + 2)),
    ]
    out_specs = [
        pl.BlockSpec((CB, 1), lambda i: (i, 0)),
        pl.BlockSpec((CB, N, 1), lambda i: (i, 0, 0)),
    ]
    user_label, post_label = pl.pallas_call(
        _main_body,
        grid=grid,
        in_specs=in_specs,
        out_specs=out_specs,
        out_shape=(
            jax.ShapeDtypeStruct((B, 1), f32),
            jax.ShapeDtypeStruct((B, N, 1), f32),
        ),
        compiler_params=pltpu.CompilerParams(
            dimension_semantics=("parallel",)),
    )(posts, post_adjs, users, mean, ginv, t, vpost, cu, wp, cp,
      Wx0, bg0, ce0, Wx1)
    return (user_label, post_label)


# speedup vs baseline: 1.2359x; 1.2359x over previous
import functools

import jax
import jax.numpy as jnp
from jax.experimental import pallas as pl
from jax.experimental.pallas import tpu as pltpu

_N = 50
_F = 128
_D = 32


def _prep_body(users_ref, gamma_ref, beta_ref, Wue_ref, bue_ref, Wpe_ref,
               bpe_ref, Wg0_ref, Wu0_ref, bu0_ref, Wu1_ref, bu1_ref,
               Wp0_ref, bp0_ref, Wp1_ref, bp1_ref,
               as0_ref, ad0_ref, Wg1_ref, as1_ref, ad1_ref,
               mean_ref, ginv_ref, t_ref, vpost_ref, cu_ref, wp_ref, cp_ref,
               Wx0_ref, vd0_ref, bg0_ref, ce0_ref, Wx1_ref, vd1_ref):
    u = users_ref[...]
    mean = jnp.mean(u, axis=0, keepdims=True)
    var = jnp.mean((u - mean) * (u - mean), axis=0, keepdims=True)
    ginv = gamma_ref[...] * jax.lax.rsqrt(var + 1e-5)
    mean_ref[...] = mean
    ginv_ref[...] = ginv

    dot = functools.partial(jnp.dot, preferred_element_type=jnp.float32)
    wu = dot(Wu0_ref[...], Wu1_ref[...])
    wu_top = wu[0:_D, :]
    t = dot(Wue_ref[...], wu_top)
    t_ref[...] = t
    vpost_ref[...] = wu[_D:2 * _D, :]
    cu_ref[...] = (dot(bu0_ref[...], Wu1_ref[...]) + bu1_ref[...]
                   + dot(beta_ref[...], t) + dot(bue_ref[...], wu_top))
    wp = dot(Wp0_ref[...], Wp1_ref[...])
    wp_ref[...] = wp
    cp_ref[...] = dot(bp0_ref[...], Wp1_ref[...]) + bp1_ref[...]
    Wg0e = dot(Wpe_ref[...], Wg0_ref[...])
    bg0 = dot(bpe_ref[...], Wg0_ref[...])
    Wx0_ref[...] = jnp.concatenate(
        [Wg0e, dot(Wg0e, as0_ref[...])], axis=1)
    vd0_ref[...] = dot(Wg0e, ad0_ref[...])
    bg0_ref[...] = bg0
    ce0_ref[...] = dot(bg0, as0_ref[...] + ad0_ref[...])
    Wg1 = Wg1_ref[...]
    Wx1_ref[...] = jnp.concatenate(
        [Wg1, dot(Wg1, as1_ref[...])], axis=1)
    vd1_ref[...] = dot(Wg1, ad1_ref[...])


def _bdot(a, b):
    return jax.lax.dot_general(
        a, b, (((2,), (1,)), ((0,), (0,))),
        preferred_element_type=jnp.float32)


def _attend(hx, x, vd, adj, ce, bias):
    cb = hx.shape[0]
    hw = hx[:, :, 0:_D] + bias
    es = hx[:, :, _D:_D + 1]
    vd_b = jnp.broadcast_to(vd.T[None], (cb, 1, x.shape[2]))
    ed = jax.lax.dot_general(
        vd_b, x, (((2,), (2,)), ((0,), (0,))),
        preferred_element_type=jnp.float32)
    e = es + ed + ce
    e = jnp.where(e >= 0, e, 0.2 * e)
    e = jnp.where(adj > 0, e, -1e9)
    m = jnp.max(e, axis=2, keepdims=True)
    p = jnp.exp(e - m)
    s = jnp.sum(p, axis=2, keepdims=True)
    out = _bdot(p, hw) / s
    return jnp.where(out > 0, out, jnp.exp(out) - 1.0)


def _main_body(posts_ref, adj_ref, users_ref, mean_ref, ginv_ref, t_ref,
               vpost_ref, cu_ref, wp_ref, cp_ref, Wx0_ref, vd0_ref, bg0_ref,
               ce0_ref, Wx1_ref, vd1_ref, ul_ref, plab_ref):
    cb = posts_ref.shape[0]
    posts = posts_ref[...]
    adj = adj_ref[...]

    Wx0_b = jnp.broadcast_to(Wx0_ref[...][None], (cb, _F, _D + 1))
    hx0 = _bdot(posts, Wx0_b)
    h1 = _attend(hx0, posts, vd0_ref[...], adj,
                 ce0_ref[...][None], bg0_ref[...][None])

    Wx1_b = jnp.broadcast_to(Wx1_ref[...][None], (cb, _D, _D + 1))
    hx1 = _bdot(h1, Wx1_b)
    zero = jnp.zeros((1, 1, 1), jnp.float32)
    pe = _attend(hx1, h1, vd1_ref[...], adj, zero, zero)

    wp_b = jnp.broadcast_to(wp_ref[...][None], (cb, _D, 1))
    pco = _bdot(pe, wp_b) + cp_ref[...][None]
    plab_ref[...] = jax.nn.sigmoid(pco)

    mp = jnp.max(pe, axis=1)
    un = (users_ref[...] - mean_ref[...]) * ginv_ref[...]
    uco = (jnp.dot(un, t_ref[...], preferred_element_type=jnp.float32)
           + jnp.dot(mp, vpost_ref[...], preferred_element_type=jnp.float32)
           + cu_ref[...])
    ul_ref[...] = jax.nn.sigmoid(uco)


def kernel(users, posts, post_adjs, up_masking, bn_gamma, bn_beta,
           W_user_enc, b_user_enc, W_post_enc, b_post_enc,
           W_gat0, a_src0, a_dst0, W_gat1, a_src1, a_dst1,
           W_pcls0, b_pcls0, W_pcls1, b_pcls1,
           W_ucls0, b_ucls0, W_ucls1, b_ucls1):
    B, F = users.shape
    N = posts.shape[1]
    D = W_gat0.shape[0]

    row = lambda v: v.reshape(1, -1)
    col = lambda v: v.reshape(-1, 1)
    f32 = jnp.float32

    prep_outs = (
        jax.ShapeDtypeStruct((1, F), f32),
        jax.ShapeDtypeStruct((1, F), f32),
        jax.ShapeDtypeStruct((F, 1), f32),
        jax.ShapeDtypeStruct((D, 1), f32),
        jax.ShapeDtypeStruct((1, 1), f32),
        jax.ShapeDtypeStruct((D, 1), f32),
        jax.ShapeDtypeStruct((1, 1), f32),
        jax.ShapeDtypeStruct((F, D + 1), f32),
        jax.ShapeDtypeStruct((F, 1), f32),
        jax.ShapeDtypeStruct((1, D), f32),
        jax.ShapeDtypeStruct((1, 1), f32),
        jax.ShapeDtypeStruct((D, D + 1), f32),
        jax.ShapeDtypeStruct((D, 1), f32),
    )
    (mean, ginv, t, vpost, cu, wp, cp, Wx0, vd0, bg0, ce0, Wx1,
     vd1) = pl.pallas_call(
        _prep_body, out_shape=prep_outs)(
            users, row(bn_gamma), row(bn_beta), W_user_enc, row(b_user_enc),
            W_post_enc, row(b_post_enc), W_gat0, W_ucls0, row(b_ucls0),
            W_ucls1, row(b_ucls1), W_pcls0, row(b_pcls0), W_pcls1,
            row(b_pcls1), col(a_src0), col(a_dst0), W_gat1,
            col(a_src1), col(a_dst1))

    CB = 64
    grid = (B // CB,)
    full = lambda shape: pl.BlockSpec(shape, lambda i: (0,) * len(shape))
    in_specs = [
        pl.BlockSpec((CB, N, F), lambda i: (i, 0, 0)),
        pl.BlockSpec((CB, N, N), lambda i: (i, 0, 0)),
        pl.BlockSpec((CB, F), lambda i: (i, 0)),
        full((1, F)), full((1, F)), full((F, 1)), full((D, 1)),
        full((1, 1)), full((D, 1)), full((1, 1)), full((F, D + 1)),
        full((F, 1)), full((1, D)), full((1, 1)), full((D, D + 1)),
        full((D, 1)),
    ]
    out_specs = [
        pl.BlockSpec((CB, 1), lambda i: (i, 0)),
        pl.BlockSpec((CB, N, 1), lambda i: (i, 0, 0)),
    ]
    user_label, post_label = pl.pallas_call(
        _main_body,
        grid=grid,
        in_specs=in_specs,
        out_specs=out_specs,
        out_shape=(
            jax.ShapeDtypeStruct((B, 1), f32),
            jax.ShapeDtypeStruct((B, N, 1), f32),
        ),
        compiler_params=pltpu.CompilerParams(
            dimension_semantics=("parallel",)),
    )(posts, post_adjs, users, mean, ginv, t, vpost, cu, wp, cp,
      Wx0, vd0, bg0, ce0, Wx1, vd1)
    return (user_label, post_label)


# speedup vs baseline: 1.2542x; 1.0148x over previous
import functools

import jax
import jax.numpy as jnp
from jax.experimental import pallas as pl
from jax.experimental.pallas import tpu as pltpu

_N = 50
_F = 128
_D = 32


def _prep_body(users_ref, gamma_ref, beta_ref, Wue_ref, bue_ref, Wpe_ref,
               bpe_ref, Wg0_ref, Wu0_ref, bu0_ref, Wu1_ref, bu1_ref,
               Wp0_ref, bp0_ref, Wp1_ref, bp1_ref,
               as0_ref, ad0_ref, Wg1_ref, as1_ref, ad1_ref,
               mean_ref, ginv_ref, t_ref, vpost_ref, cu_ref, wp_ref, cp_ref,
               Wx0_ref, vd0_ref, bg0_ref, ce0_ref, Wx1_ref, vd1_ref):
    u = users_ref[...]
    mean = jnp.mean(u, axis=0, keepdims=True)
    var = jnp.mean((u - mean) * (u - mean), axis=0, keepdims=True)
    ginv = gamma_ref[...] * jax.lax.rsqrt(var + 1e-5)
    mean_ref[...] = mean
    ginv_ref[...] = ginv

    dot = functools.partial(jnp.dot, preferred_element_type=jnp.float32)
    wu = dot(Wu0_ref[...], Wu1_ref[...])
    wu_top = wu[0:_D, :]
    t = dot(Wue_ref[...], wu_top)
    t_ref[...] = t
    vpost_ref[...] = wu[_D:2 * _D, :]
    cu_ref[...] = (dot(bu0_ref[...], Wu1_ref[...]) + bu1_ref[...]
                   + dot(beta_ref[...], t) + dot(bue_ref[...], wu_top))
    wp = dot(Wp0_ref[...], Wp1_ref[...])
    wp_ref[...] = wp
    cp_ref[...] = dot(bp0_ref[...], Wp1_ref[...]) + bp1_ref[...]
    Wg0e = dot(Wpe_ref[...], Wg0_ref[...])
    bg0 = dot(bpe_ref[...], Wg0_ref[...])
    Wx0_ref[...] = jnp.concatenate(
        [Wg0e, dot(Wg0e, as0_ref[...])], axis=1)
    vd0_ref[...] = dot(Wg0e, ad0_ref[...])
    bg0_ref[...] = bg0
    ce0_ref[...] = dot(bg0, as0_ref[...] + ad0_ref[...])
    Wg1 = Wg1_ref[...]
    Wx1_ref[...] = jnp.concatenate(
        [Wg1, dot(Wg1, as1_ref[...])], axis=1)
    vd1_ref[...] = dot(Wg1, ad1_ref[...])


def _bdot(a, b):
    return jax.lax.dot_general(
        a, b, (((2,), (1,)), ((0,), (0,))),
        preferred_element_type=jnp.float32)


def _attend(hx, x, vd, adj, ce, bias):
    cb = hx.shape[0]
    hw = hx[:, :, 0:_D] + bias
    es = hx[:, :, _D:_D + 1]
    vd_b = jnp.broadcast_to(vd.T[None], (cb, 1, x.shape[2]))
    ed = jax.lax.dot_general(
        vd_b, x, (((2,), (2,)), ((0,), (0,))),
        preferred_element_type=jnp.float32)
    e = es + ed + ce
    e = jnp.where(e >= 0, e, 0.2 * e)
    e = jnp.where(adj > 0, e, -1e9)
    m = jnp.max(e, axis=2, keepdims=True)
    p = jnp.exp(e - m)
    s = jnp.sum(p, axis=2, keepdims=True)
    out = _bdot(p, hw) / s
    return jnp.where(out > 0, out, jnp.exp(out) - 1.0)


def _main_body(posts_ref, adj_ref, users_ref, mean_ref, ginv_ref, t_ref,
               vpost_ref, cu_ref, wp_ref, cp_ref, Wx0_ref, vd0_ref, bg0_ref,
               ce0_ref, Wx1_ref, vd1_ref, ul_ref, plab_ref):
    cb = posts_ref.shape[0]
    posts = posts_ref[...]
    adj = adj_ref[...]

    Wx0_b = jnp.broadcast_to(Wx0_ref[...][None], (cb, _F, _D + 1))
    hx0 = _bdot(posts, Wx0_b)
    h1 = _attend(hx0, posts, vd0_ref[...], adj,
                 ce0_ref[...][None], bg0_ref[...][None])

    Wx1_b = jnp.broadcast_to(Wx1_ref[...][None], (cb, _D, _D + 1))
    hx1 = _bdot(h1, Wx1_b)
    zero = jnp.zeros((1, 1, 1), jnp.float32)
    pe = _attend(hx1, h1, vd1_ref[...], adj, zero, zero)

    wp_b = jnp.broadcast_to(wp_ref[...][None], (cb, _D, 1))
    pco = _bdot(pe, wp_b) + cp_ref[...][None]
    plab_ref[...] = jax.nn.sigmoid(pco)

    mp = jnp.max(pe, axis=1)
    un = (users_ref[...] - mean_ref[...]) * ginv_ref[...]
    uco = (jnp.dot(un, t_ref[...], preferred_element_type=jnp.float32)
           + jnp.dot(mp, vpost_ref[...], preferred_element_type=jnp.float32)
           + cu_ref[...])
    ul_ref[...] = jax.nn.sigmoid(uco)


def kernel(users, posts, post_adjs, up_masking, bn_gamma, bn_beta,
           W_user_enc, b_user_enc, W_post_enc, b_post_enc,
           W_gat0, a_src0, a_dst0, W_gat1, a_src1, a_dst1,
           W_pcls0, b_pcls0, W_pcls1, b_pcls1,
           W_ucls0, b_ucls0, W_ucls1, b_ucls1):
    B, F = users.shape
    N = posts.shape[1]
    D = W_gat0.shape[0]

    row = lambda v: v.reshape(1, -1)
    col = lambda v: v.reshape(-1, 1)
    f32 = jnp.float32

    prep_outs = (
        jax.ShapeDtypeStruct((1, F), f32),
        jax.ShapeDtypeStruct((1, F), f32),
        jax.ShapeDtypeStruct((F, 1), f32),
        jax.ShapeDtypeStruct((D, 1), f32),
        jax.ShapeDtypeStruct((1, 1), f32),
        jax.ShapeDtypeStruct((D, 1), f32),
        jax.ShapeDtypeStruct((1, 1), f32),
        jax.ShapeDtypeStruct((F, D + 1), f32),
        jax.ShapeDtypeStruct((F, 1), f32),
        jax.ShapeDtypeStruct((1, D), f32),
        jax.ShapeDtypeStruct((1, 1), f32),
        jax.ShapeDtypeStruct((D, D + 1), f32),
        jax.ShapeDtypeStruct((D, 1), f32),
    )
    (mean, ginv, t, vpost, cu, wp, cp, Wx0, vd0, bg0, ce0, Wx1,
     vd1) = pl.pallas_call(
        _prep_body, out_shape=prep_outs)(
            users, row(bn_gamma), row(bn_beta), W_user_enc, row(b_user_enc),
            W_post_enc, row(b_post_enc), W_gat0, W_ucls0, row(b_ucls0),
            W_ucls1, row(b_ucls1), W_pcls0, row(b_pcls0), W_pcls1,
            row(b_pcls1), col(a_src0), col(a_dst0), W_gat1,
            col(a_src1), col(a_dst1))

    CB = 128
    grid = (B // CB,)
    full = lambda shape: pl.BlockSpec(shape, lambda i: (0,) * len(shape))
    in_specs = [
        pl.BlockSpec((CB, N, F), lambda i: (i, 0, 0)),
        pl.BlockSpec((CB, N, N), lambda i: (i, 0, 0)),
        pl.BlockSpec((CB, F), lambda i: (i, 0)),
        full((1, F)), full((1, F)), full((F, 1)), full((D, 1)),
        full((1, 1)), full((D, 1)), full((1, 1)), full((F, D + 1)),
        full((F, 1)), full((1, D)), full((1, 1)), full((D, D + 1)),
        full((D, 1)),
    ]
    out_specs = [
        pl.BlockSpec((CB, 1), lambda i: (i, 0)),
        pl.BlockSpec((CB, N, 1), lambda i: (i, 0, 0)),
    ]
    user_label, post_label = pl.pallas_call(
        _main_body,
        grid=grid,
        in_specs=in_specs,
        out_specs=out_specs,
        out_shape=(
            jax.ShapeDtypeStruct((B, 1), f32),
            jax.ShapeDtypeStruct((B, N, 1), f32),
        ),
        compiler_params=pltpu.CompilerParams(
            dimension_semantics=("parallel",)),
    )(posts, post_adjs, users, mean, ginv, t, vpost, cu, wp, cp,
      Wx0, vd0, bg0, ce0, Wx1, vd1)
    return (user_label, post_label)


# speedup vs baseline: 1.2765x; 1.0178x over previous
import functools

import jax
import jax.numpy as jnp
from jax.experimental import pallas as pl
from jax.experimental.pallas import tpu as pltpu

_N = 50
_F = 128
_D = 32


def _prep_body(users_ref, gamma_ref, beta_ref, Wue_ref, bue_ref, Wpe_ref,
               bpe_ref, Wg0_ref, Wu0_ref, bu0_ref, Wu1_ref, bu1_ref,
               Wp0_ref, bp0_ref, Wp1_ref, bp1_ref,
               as0_ref, ad0_ref, Wg1_ref, as1_ref, ad1_ref,
               mean_ref, ginv_ref, t_ref, vpost_ref, cu_ref, wp_ref, cp_ref,
               Wx0_ref, vd0_ref, bg0_ref, ce0_ref, Wx1_ref, vd1_ref):
    u = users_ref[...]
    mean = jnp.mean(u, axis=0, keepdims=True)
    var = jnp.mean((u - mean) * (u - mean), axis=0, keepdims=True)
    ginv = gamma_ref[...] * jax.lax.rsqrt(var + 1e-5)
    mean_ref[...] = mean
    ginv_ref[...] = ginv

    dot = functools.partial(jnp.dot, preferred_element_type=jnp.float32)
    wu = dot(Wu0_ref[...], Wu1_ref[...])
    wu_top = wu[0:_D, :]
    t = dot(Wue_ref[...], wu_top)
    t_ref[...] = t
    vpost_ref[...] = wu[_D:2 * _D, :]
    cu_ref[...] = (dot(bu0_ref[...], Wu1_ref[...]) + bu1_ref[...]
                   + dot(beta_ref[...], t) + dot(bue_ref[...], wu_top))
    wp = dot(Wp0_ref[...], Wp1_ref[...])
    wp_ref[...] = wp
    cp_ref[...] = dot(bp0_ref[...], Wp1_ref[...]) + bp1_ref[...]
    Wg0e = dot(Wpe_ref[...], Wg0_ref[...])
    bg0 = dot(bpe_ref[...], Wg0_ref[...])
    Wx0_ref[...] = jnp.concatenate(
        [Wg0e, dot(Wg0e, as0_ref[...])], axis=1)
    vd0_ref[...] = dot(Wg0e, ad0_ref[...])
    bg0_ref[...] = bg0
    ce0_ref[...] = dot(bg0, as0_ref[...] + ad0_ref[...])
    Wg1 = Wg1_ref[...]
    Wx1_ref[...] = jnp.concatenate(
        [Wg1, dot(Wg1, as1_ref[...])], axis=1)
    vd1_ref[...] = dot(Wg1, ad1_ref[...])


def _bdot(a, b):
    return jax.lax.dot_general(
        a.astype(jnp.bfloat16), b.astype(jnp.bfloat16),
        (((2,), (1,)), ((0,), (0,))),
        preferred_element_type=jnp.float32)


def _attend(hx, x, vd, adj, ce, bias):
    cb = hx.shape[0]
    hw = hx[:, :, 0:_D] + bias
    es = hx[:, :, _D:_D + 1]
    vd_b = jnp.broadcast_to(vd.T[None], (cb, 1, x.shape[2]))
    ed = jax.lax.dot_general(
        vd_b.astype(jnp.bfloat16), x.astype(jnp.bfloat16),
        (((2,), (2,)), ((0,), (0,))),
        preferred_element_type=jnp.float32)
    e = es + ed + ce
    e = jnp.where(e >= 0, e, 0.2 * e)
    e = jnp.where(adj > 0, e, -1e9)
    m = jnp.max(e, axis=2, keepdims=True)
    p = jnp.exp(e - m)
    s = jnp.sum(p, axis=2, keepdims=True)
    out = _bdot(p, hw) / s
    return jnp.where(out > 0, out, jnp.exp(out) - 1.0)


def _main_body(posts_ref, adj_ref, users_ref, mean_ref, ginv_ref, t_ref,
               vpost_ref, cu_ref, wp_ref, cp_ref, Wx0_ref, vd0_ref, bg0_ref,
               ce0_ref, Wx1_ref, vd1_ref, ul_ref, plab_ref):
    cb = posts_ref.shape[0]
    posts = posts_ref[...]
    adj = adj_ref[...]

    Wx0_b = jnp.broadcast_to(Wx0_ref[...][None], (cb, _F, _D + 1))
    hx0 = _bdot(posts, Wx0_b)
    h1 = _attend(hx0, posts, vd0_ref[...], adj,
                 ce0_ref[...][None], bg0_ref[...][None])

    Wx1_b = jnp.broadcast_to(Wx1_ref[...][None], (cb, _D, _D + 1))
    hx1 = _bdot(h1, Wx1_b)
    zero = jnp.zeros((1, 1, 1), jnp.float32)
    pe = _attend(hx1, h1, vd1_ref[...], adj, zero, zero)

    wp_b = jnp.broadcast_to(wp_ref[...][None], (cb, _D, 1))
    pco = _bdot(pe, wp_b) + cp_ref[...][None]
    plab_ref[...] = jax.nn.sigmoid(pco)

    mp = jnp.max(pe, axis=1)
    un = (users_ref[...] - mean_ref[...]) * ginv_ref[...]
    uco = (jnp.dot(un, t_ref[...], preferred_element_type=jnp.float32)
           + jnp.dot(mp, vpost_ref[...], preferred_element_type=jnp.float32)
           + cu_ref[...])
    ul_ref[...] = jax.nn.sigmoid(uco)


def kernel(users, posts, post_adjs, up_masking, bn_gamma, bn_beta,
           W_user_enc, b_user_enc, W_post_enc, b_post_enc,
           W_gat0, a_src0, a_dst0, W_gat1, a_src1, a_dst1,
           W_pcls0, b_pcls0, W_pcls1, b_pcls1,
           W_ucls0, b_ucls0, W_ucls1, b_ucls1):
    B, F = users.shape
    N = posts.shape[1]
    D = W_gat0.shape[0]

    row = lambda v: v.reshape(1, -1)
    col = lambda v: v.reshape(-1, 1)
    f32 = jnp.float32

    prep_outs = (
        jax.ShapeDtypeStruct((1, F), f32),
        jax.ShapeDtypeStruct((1, F), f32),
        jax.ShapeDtypeStruct((F, 1), f32),
        jax.ShapeDtypeStruct((D, 1), f32),
        jax.ShapeDtypeStruct((1, 1), f32),
        jax.ShapeDtypeStruct((D, 1), f32),
        jax.ShapeDtypeStruct((1, 1), f32),
        jax.ShapeDtypeStruct((F, D + 1), f32),
        jax.ShapeDtypeStruct((F, 1), f32),
        jax.ShapeDtypeStruct((1, D), f32),
        jax.ShapeDtypeStruct((1, 1), f32),
        jax.ShapeDtypeStruct((D, D + 1), f32),
        jax.ShapeDtypeStruct((D, 1), f32),
    )
    (mean, ginv, t, vpost, cu, wp, cp, Wx0, vd0, bg0, ce0, Wx1,
     vd1) = pl.pallas_call(
        _prep_body, out_shape=prep_outs)(
            users, row(bn_gamma), row(bn_beta), W_user_enc, row(b_user_enc),
            W_post_enc, row(b_post_enc), W_gat0, W_ucls0, row(b_ucls0),
            W_ucls1, row(b_ucls1), W_pcls0, row(b_pcls0), W_pcls1,
            row(b_pcls1), col(a_src0), col(a_dst0), W_gat1,
            col(a_src1), col(a_dst1))

    CB = 128
    grid = (B // CB,)
    full = lambda shape: pl.BlockSpec(shape, lambda i: (0,) * len(shape))
    in_specs = [
        pl.BlockSpec((CB, N, F), lambda i: (i, 0, 0)),
        pl.BlockSpec((CB, N, N), lambda i: (i, 0, 0)),
        pl.BlockSpec((CB, F), lambda i: (i, 0)),
        full((1, F)), full((1, F)), full((F, 1)), full((D, 1)),
        full((1, 1)), full((D, 1)), full((1, 1)), full((F, D + 1)),
        full((F, 1)), full((1, D)), full((1, 1)), full((D, D + 1)),
        full((D, 1)),
    ]
    out_specs = [
        pl.BlockSpec((CB, 1), lambda i: (i, 0)),
        pl.BlockSpec((CB, N, 1), lambda i: (i, 0, 0)),
    ]
    user_label, post_label = pl.pallas_call(
        _main_body,
        grid=grid,
        in_specs=in_specs,
        out_specs=out_specs,
        out_shape=(
            jax.ShapeDtypeStruct((B, 1), f32),
            jax.ShapeDtypeStruct((B, N, 1), f32),
        ),
        compiler_params=pltpu.CompilerParams(
            dimension_semantics=("parallel",)),
    )(posts, post_adjs, users, mean, ginv, t, vpost, cu, wp, cp,
      Wx0, vd0, bg0, ce0, Wx1, vd1)
    return (user_label, post_label)


# speedup vs baseline: 1.3408x; 1.0503x over previous
import functools

import jax
import jax.numpy as jnp
from jax.experimental import pallas as pl
from jax.experimental.pallas import tpu as pltpu

_N = 50
_F = 128
_D = 32


def _prep_body(users_ref, gamma_ref, beta_ref, Wue_ref, bue_ref, Wpe_ref,
               bpe_ref, Wg0_ref, Wu0_ref, bu0_ref, Wu1_ref, bu1_ref,
               Wp0_ref, bp0_ref, Wp1_ref, bp1_ref,
               as0_ref, ad0_ref, Wg1_ref, as1_ref, ad1_ref,
               mean_ref, ginv_ref, t_ref, vpost_ref, cu_ref, wp_ref, cp_ref,
               Wx0_ref, vd0_ref, bg0_ref, ce0_ref, Wx1_ref, vd1_ref):
    u = users_ref[...]
    mean = jnp.mean(u, axis=0, keepdims=True)
    var = jnp.mean((u - mean) * (u - mean), axis=0, keepdims=True)
    ginv = gamma_ref[...] * jax.lax.rsqrt(var + 1e-5)
    mean_ref[...] = mean
    ginv_ref[...] = ginv

    dot = functools.partial(jnp.dot, preferred_element_type=jnp.float32)
    wu = dot(Wu0_ref[...], Wu1_ref[...])
    wu_top = wu[0:_D, :]
    t = dot(Wue_ref[...], wu_top)
    t_ref[...] = t
    vpost_ref[...] = wu[_D:2 * _D, :]
    cu_ref[...] = (dot(bu0_ref[...], Wu1_ref[...]) + bu1_ref[...]
                   + dot(beta_ref[...], t) + dot(bue_ref[...], wu_top))
    wp = dot(Wp0_ref[...], Wp1_ref[...])
    wp_ref[...] = wp
    cp_ref[...] = dot(bp0_ref[...], Wp1_ref[...]) + bp1_ref[...]
    Wg0e = dot(Wpe_ref[...], Wg0_ref[...])
    bg0 = dot(bpe_ref[...], Wg0_ref[...])
    Wx0_ref[...] = jnp.concatenate(
        [Wg0e, dot(Wg0e, as0_ref[...])], axis=1)
    vd0_ref[...] = dot(Wg0e, ad0_ref[...])
    bg0_ref[...] = bg0
    ce0_ref[...] = dot(bg0, as0_ref[...] + ad0_ref[...])
    Wg1 = Wg1_ref[...]
    Wx1_ref[...] = jnp.concatenate(
        [Wg1, dot(Wg1, as1_ref[...])], axis=1)
    vd1_ref[...] = dot(Wg1, ad1_ref[...])


def _bdot(a, b):
    return jax.lax.dot_general(
        a.astype(jnp.bfloat16), b.astype(jnp.bfloat16),
        (((2,), (1,)), ((0,), (0,))),
        preferred_element_type=jnp.float32)


def _attend(hx, x, vd, adj, ce, bias):
    cb = hx.shape[0]
    hw = hx[:, :, 0:_D] + bias
    es = hx[:, :, _D:_D + 1]
    vd_b = jnp.broadcast_to(vd.T[None], (cb, 1, x.shape[2]))
    ed = jax.lax.dot_general(
        vd_b.astype(jnp.bfloat16), x.astype(jnp.bfloat16),
        (((2,), (2,)), ((0,), (0,))),
        preferred_element_type=jnp.float32)
    e = es + (ed + ce)
    e = jnp.maximum(e, 0.2 * e)
    p = jnp.where(adj > 0, jnp.exp(e), 0.0)
    s = jnp.sum(p, axis=2, keepdims=True)
    out = _bdot(p, hw) / s
    return jnp.where(out > 0, out, jnp.exp(out) - 1.0)


def _main_body(posts_ref, adj_ref, users_ref, mean_ref, ginv_ref, t_ref,
               vpost_ref, cu_ref, wp_ref, cp_ref, Wx0_ref, vd0_ref, bg0_ref,
               ce0_ref, Wx1_ref, vd1_ref, ul_ref, plab_ref):
    cb = posts_ref.shape[0]
    posts = posts_ref[...]
    adj = adj_ref[...]

    Wx0_b = jnp.broadcast_to(Wx0_ref[...][None], (cb, _F, _D + 1))
    hx0 = _bdot(posts, Wx0_b)
    h1 = _attend(hx0, posts, vd0_ref[...], adj,
                 ce0_ref[...][None], bg0_ref[...][None])

    Wx1_b = jnp.broadcast_to(Wx1_ref[...][None], (cb, _D, _D + 1))
    hx1 = _bdot(h1, Wx1_b)
    zero = jnp.zeros((1, 1, 1), jnp.float32)
    pe = _attend(hx1, h1, vd1_ref[...], adj, zero, zero)

    wp_b = jnp.broadcast_to(wp_ref[...][None], (cb, _D, 1))
    pco = _bdot(pe, wp_b) + cp_ref[...][None]
    plab_ref[...] = jax.nn.sigmoid(pco)

    mp = jnp.max(pe, axis=1)
    un = (users_ref[...] - mean_ref[...]) * ginv_ref[...]
    uco = (jnp.dot(un, t_ref[...], preferred_element_type=jnp.float32)
           + jnp.dot(mp, vpost_ref[...], preferred_element_type=jnp.float32)
           + cu_ref[...])
    ul_ref[...] = jax.nn.sigmoid(uco)


def kernel(users, posts, post_adjs, up_masking, bn_gamma, bn_beta,
           W_user_enc, b_user_enc, W_post_enc, b_post_enc,
           W_gat0, a_src0, a_dst0, W_gat1, a_src1, a_dst1,
           W_pcls0, b_pcls0, W_pcls1, b_pcls1,
           W_ucls0, b_ucls0, W_ucls1, b_ucls1):
    B, F = users.shape
    N = posts.shape[1]
    D = W_gat0.shape[0]

    row = lambda v: v.reshape(1, -1)
    col = lambda v: v.reshape(-1, 1)
    f32 = jnp.float32

    prep_outs = (
        jax.ShapeDtypeStruct((1, F), f32),
        jax.ShapeDtypeStruct((1, F), f32),
        jax.ShapeDtypeStruct((F, 1), f32),
        jax.ShapeDtypeStruct((D, 1), f32),
        jax.ShapeDtypeStruct((1, 1), f32),
        jax.ShapeDtypeStruct((D, 1), f32),
        jax.ShapeDtypeStruct((1, 1), f32),
        jax.ShapeDtypeStruct((F, D + 1), f32),
        jax.ShapeDtypeStruct((F, 1), f32),
        jax.ShapeDtypeStruct((1, D), f32),
        jax.ShapeDtypeStruct((1, 1), f32),
        jax.ShapeDtypeStruct((D, D + 1), f32),
        jax.ShapeDtypeStruct((D, 1), f32),
    )
    (mean, ginv, t, vpost, cu, wp, cp, Wx0, vd0, bg0, ce0, Wx1,
     vd1) = pl.pallas_call(
        _prep_body, out_shape=prep_outs)(
            users, row(bn_gamma), row(bn_beta), W_user_enc, row(b_user_enc),
            W_post_enc, row(b_post_enc), W_gat0, W_ucls0, row(b_ucls0),
            W_ucls1, row(b_ucls1), W_pcls0, row(b_pcls0), W_pcls1,
            row(b_pcls1), col(a_src0), col(a_dst0), W_gat1,
            col(a_src1), col(a_dst1))

    CB = 128
    grid = (B // CB,)
    full = lambda shape: pl.BlockSpec(shape, lambda i: (0,) * len(shape))
    in_specs = [
        pl.BlockSpec((CB, N, F), lambda i: (i, 0, 0)),
        pl.BlockSpec((CB, N, N), lambda i: (i, 0, 0)),
        pl.BlockSpec((CB, F), lambda i: (i, 0)),
        full((1, F)), full((1, F)), full((F, 1)), full((D, 1)),
        full((1, 1)), full((D, 1)), full((1, 1)), full((F, D + 1)),
        full((F, 1)), full((1, D)), full((1, 1)), full((D, D + 1)),
        full((D, 1)),
    ]
    out_specs = [
        pl.BlockSpec((CB, 1), lambda i: (i, 0)),
        pl.BlockSpec((CB, N, 1), lambda i: (i, 0, 0)),
    ]
    user_label, post_label = pl.pallas_call(
        _main_body,
        grid=grid,
        in_specs=in_specs,
        out_specs=out_specs,
        out_shape=(
            jax.ShapeDtypeStruct((B, 1), f32),
            jax.ShapeDtypeStruct((B, N, 1), f32),
        ),
        compiler_params=pltpu.CompilerParams(
            dimension_semantics=("parallel",)),
    )(posts, post_adjs, users, mean, ginv, t, vpost, cu, wp, cp,
      Wx0, vd0, bg0, ce0, Wx1, vd1)
    return (user_label, post_label)


# speedup vs baseline: 1.3487x; 1.0059x over previous
import functools

import jax
import jax.numpy as jnp
from jax.experimental import pallas as pl
from jax.experimental.pallas import tpu as pltpu

_N = 50
_F = 128
_D = 32


def _prep_body(users_ref, gamma_ref, beta_ref, Wue_ref, bue_ref, Wpe_ref,
               bpe_ref, Wg0_ref, Wu0_ref, bu0_ref, Wu1_ref, bu1_ref,
               Wp0_ref, bp0_ref, Wp1_ref, bp1_ref,
               as0_ref, ad0_ref, Wg1_ref, as1_ref, ad1_ref,
               mean_ref, ginv_ref, t_ref, vpost_ref, cu_ref, wp_ref, cp_ref,
               Wx0_ref, vd0_ref, bg0_ref, ce0_ref, Wx1_ref, vd1_ref):
    u = users_ref[...]
    mean = jnp.mean(u, axis=0, keepdims=True)
    var = jnp.mean((u - mean) * (u - mean), axis=0, keepdims=True)
    ginv = gamma_ref[...] * jax.lax.rsqrt(var + 1e-5)
    mean_ref[...] = mean
    ginv_ref[...] = ginv

    dot = functools.partial(jnp.dot, preferred_element_type=jnp.float32)
    wu = dot(Wu0_ref[...], Wu1_ref[...])
    wu_top = wu[0:_D, :]
    t = dot(Wue_ref[...], wu_top)
    t_ref[...] = t
    vpost_ref[...] = wu[_D:2 * _D, :]
    cu_ref[...] = (dot(bu0_ref[...], Wu1_ref[...]) + bu1_ref[...]
                   + dot(beta_ref[...], t) + dot(bue_ref[...], wu_top))
    wp = dot(Wp0_ref[...], Wp1_ref[...])
    wp_ref[...] = wp
    cp_ref[...] = dot(bp0_ref[...], Wp1_ref[...]) + bp1_ref[...]
    Wg0e = dot(Wpe_ref[...], Wg0_ref[...])
    bg0 = dot(bpe_ref[...], Wg0_ref[...])
    Wx0_ref[...] = jnp.concatenate(
        [Wg0e, dot(Wg0e, as0_ref[...])], axis=1)
    vd0_ref[...] = dot(Wg0e, ad0_ref[...])
    bg0_ref[...] = bg0
    ce0_ref[...] = dot(bg0, as0_ref[...] + ad0_ref[...])
    Wg1 = Wg1_ref[...]
    Wx1_ref[...] = jnp.concatenate(
        [Wg1, dot(Wg1, as1_ref[...])], axis=1)
    vd1_ref[...] = dot(Wg1, ad1_ref[...])


_BF = jnp.bfloat16


def _bdot(a, b, out_dtype=jnp.float32):
    r = jax.lax.dot_general(
        a.astype(_BF), b.astype(_BF),
        (((2,), (1,)), ((0,), (0,))),
        preferred_element_type=jnp.float32)
    return r.astype(out_dtype)


def _attend(hx, x, vd, adj, ce, bias):
    cb = hx.shape[0]
    es = hx[:, :, _D:_D + 1]
    vd_b = jnp.broadcast_to(vd.T[None], (cb, 1, x.shape[2]))
    ed = jax.lax.dot_general(
        vd_b.astype(_BF), x.astype(_BF),
        (((2,), (2,)), ((0,), (0,))),
        preferred_element_type=jnp.float32).astype(_BF)
    e = es + (ed + ce.astype(_BF))
    e = jnp.maximum(e, _BF(0.2) * e)
    p = jnp.where(adj > 0, jnp.exp(e), _BF(0.0))
    hwo = jnp.concatenate(
        [hx[:, :, 0:_D] + bias.astype(_BF),
         jnp.ones((cb, _N, 1), _BF)], axis=2)
    oext = _bdot(p, hwo)
    out = oext[:, :, 0:_D] / oext[:, :, _D:_D + 1]
    return jnp.where(out > 0, out, jnp.exp(out) - 1.0)


def _main_body(posts_ref, adj_ref, users_ref, mean_ref, ginv_ref, t_ref,
               vpost_ref, cu_ref, wp_ref, cp_ref, Wx0_ref, vd0_ref, bg0_ref,
               ce0_ref, Wx1_ref, vd1_ref, ul_ref, plab_ref):
    cb = posts_ref.shape[0]
    posts = posts_ref[...]
    adj = adj_ref[...]

    Wx0_b = jnp.broadcast_to(Wx0_ref[...][None], (cb, _F, _D + 1))
    hx0 = _bdot(posts, Wx0_b, out_dtype=_BF)
    h1 = _attend(hx0, posts, vd0_ref[...], adj,
                 ce0_ref[...][None], bg0_ref[...][None])

    Wx1_b = jnp.broadcast_to(Wx1_ref[...][None], (cb, _D, _D + 1))
    hx1 = _bdot(h1, Wx1_b, out_dtype=_BF)
    zero = jnp.zeros((1, 1, 1), jnp.float32)
    pe = _attend(hx1, h1, vd1_ref[...], adj, zero, zero)

    wp_b = jnp.broadcast_to(wp_ref[...][None], (cb, _D, 1))
    pco = _bdot(pe, wp_b) + cp_ref[...][None]
    plab_ref[...] = jax.nn.sigmoid(pco)

    mp = jnp.max(pe, axis=1)
    un = (users_ref[...] - mean_ref[...]) * ginv_ref[...]
    uco = (jnp.dot(un, t_ref[...], preferred_element_type=jnp.float32)
           + jnp.dot(mp, vpost_ref[...], preferred_element_type=jnp.float32)
           + cu_ref[...])
    ul_ref[...] = jax.nn.sigmoid(uco)


def kernel(users, posts, post_adjs, up_masking, bn_gamma, bn_beta,
           W_user_enc, b_user_enc, W_post_enc, b_post_enc,
           W_gat0, a_src0, a_dst0, W_gat1, a_src1, a_dst1,
           W_pcls0, b_pcls0, W_pcls1, b_pcls1,
           W_ucls0, b_ucls0, W_ucls1, b_ucls1):
    B, F = users.shape
    N = posts.shape[1]
    D = W_gat0.shape[0]

    row = lambda v: v.reshape(1, -1)
    col = lambda v: v.reshape(-1, 1)
    f32 = jnp.float32

    prep_outs = (
        jax.ShapeDtypeStruct((1, F), f32),
        jax.ShapeDtypeStruct((1, F), f32),
        jax.ShapeDtypeStruct((F, 1), f32),
        jax.ShapeDtypeStruct((D, 1), f32),
        jax.ShapeDtypeStruct((1, 1), f32),
        jax.ShapeDtypeStruct((D, 1), f32),
        jax.ShapeDtypeStruct((1, 1), f32),
        jax.ShapeDtypeStruct((F, D + 1), f32),
        jax.ShapeDtypeStruct((F, 1), f32),
        jax.ShapeDtypeStruct((1, D), f32),
        jax.ShapeDtypeStruct((1, 1), f32),
        jax.ShapeDtypeStruct((D, D + 1), f32),
        jax.ShapeDtypeStruct((D, 1), f32),
    )
    (mean, ginv, t, vpost, cu, wp, cp, Wx0, vd0, bg0, ce0, Wx1,
     vd1) = pl.pallas_call(
        _prep_body, out_shape=prep_outs)(
            users, row(bn_gamma), row(bn_beta), W_user_enc, row(b_user_enc),
            W_post_enc, row(b_post_enc), W_gat0, W_ucls0, row(b_ucls0),
            W_ucls1, row(b_ucls1), W_pcls0, row(b_pcls0), W_pcls1,
            row(b_pcls1), col(a_src0), col(a_dst0), W_gat1,
            col(a_src1), col(a_dst1))

    CB = 128
    grid = (B // CB,)
    full = lambda shape: pl.BlockSpec(shape, lambda i: (0,) * len(shape))
    in_specs = [
        pl.BlockSpec((CB, N, F), lambda i: (i, 0, 0)),
        pl.BlockSpec((CB, N, N), lambda i: (i, 0, 0)),
        pl.BlockSpec((CB, F), lambda i: (i, 0)),
        full((1, F)), full((1, F)), full((F, 1)), full((D, 1)),
        full((1, 1)), full((D, 1)), full((1, 1)), full((F, D + 1)),
        full((F, 1)), full((1, D)), full((1, 1)), full((D, D + 1)),
        full((D, 1)),
    ]
    out_specs = [
        pl.BlockSpec((CB, 1), lambda i: (i, 0)),
        pl.BlockSpec((CB, N, 1), lambda i: (i, 0, 0)),
    ]
    user_label, post_label = pl.pallas_call(
        _main_body,
        grid=grid,
        in_specs=in_specs,
        out_specs=out_specs,
        out_shape=(
            jax.ShapeDtypeStruct((B, 1), f32),
            jax.ShapeDtypeStruct((B, N, 1), f32),
        ),
        compiler_params=pltpu.CompilerParams(
            dimension_semantics=("parallel",)),
    )(posts, post_adjs, users, mean, ginv, t, vpost, cu, wp, cp,
      Wx0, vd0, bg0, ce0, Wx1, vd1)
    return (user_label, post_label)
